# baseline (speedup 1.0000x reference)
import functools
import math

import jax
import jax.numpy as jnp
import numpy as np
from jax import lax
from jax.experimental import pallas as pl
from jax.experimental.pallas import tpu as pltpu
from jax.experimental.pallas import tpu_sc as plsc

SPACE_DIMS = 3
N_SEG = 16
ROW = 2048
LANES = 16
NITER = ROW // LANES
NC = 2
NS = 16
SEG_PER_CORE = N_SEG // NC

def _deg2dist(deg):
    return 2.0 * np.sin(np.radians(np.asarray(deg, dtype=np.float64)) / 2.0)

_T_MIN = np.float32(_deg2dist(10.0 / 3600.0) ** 2)
_T_MAX = np.float32(_deg2dist(1.0) ** 2)
_LOG_RANGE = np.float32(np.log(np.float64(_T_MAX) / np.float64(_T_MIN)))
_SIGMA = np.float32(np.e)
_INV_SIGMA = np.float32(1.0) / _SIGMA
_COEF = np.float32(np.float32(1.0 / np.sqrt(2.0 * np.pi)) / _SIGMA)
_LN2 = np.float32(0.693147180559945309)
_LOG_1EM30 = np.float32(np.log(1e-30))


def _logf(x):
    ix = lax.bitcast_convert_type(x, jnp.int32)
    ix = ix + (0x3F800000 - 0x3F3504F3)
    k = lax.shift_right_arithmetic(ix, 23) - 127
    mx = (ix & 0x007FFFFF) + 0x3F3504F3
    m = lax.bitcast_convert_type(mx, jnp.float32)
    f = m - 1.0
    s = f / (2.0 + f)
    z = s * s
    w = z * z
    t1 = w * (np.float32(0.40000972152) + w * np.float32(0.24279078841))
    t2 = z * (np.float32(0.66666662693) + w * np.float32(0.28498786688))
    r = t2 + t1
    hfsq = np.float32(0.5) * f * f
    return f - (hfsq - s * (hfsq + r)) + k.astype(jnp.float32) * _LN2


@functools.partial(
    pl.kernel,
    out_type=(
        jax.ShapeDtypeStruct((N_SEG,), jnp.float32),
        jax.ShapeDtypeStruct((N_SEG,), jnp.float32),
        jax.ShapeDtypeStruct((N_SEG,), jnp.float32),
    ),
    mesh=plsc.VectorSubcoreMesh(
        core_axis_name="c", subcore_axis_name="s", num_cores=NC, num_subcores=NS
    ),
    compiler_params=pltpu.CompilerParams(needs_layout_passes=False),
    scratch_types=[
        pltpu.VMEM((2, ROW // 2), jnp.float32),
        pltpu.VMEM((2, ROW // 2), jnp.float32),
        pltpu.VMEM((2, ROW // 2), jnp.float32),
        pltpu.VMEM((2, ROW // 2), jnp.float32),
        pltpu.VMEM((2, ROW // 2), jnp.float32),
        pltpu.VMEM((2, ROW // 2), jnp.float32),
        pltpu.VMEM((2, ROW // 2), jnp.float32),
        pltpu.VMEM((2, ROW // 2), jnp.float32),
        pltpu.VMEM((LANES,), jnp.float32),
        pltpu.VMEM((ROW,), jnp.float32),
        pltpu.VMEM((ROW,), jnp.float32),
        pltpu.VMEM((LANES,), jnp.float32),
        pltpu.VMEM((LANES,), jnp.float32),
        pltpu.VMEM((LANES,), jnp.float32),
        pltpu.VMEM((SEG_PER_CORE, LANES), jnp.float32),
        pltpu.VMEM((LANES,), jnp.float32),
        pltpu.HBM((3, N_SEG, LANES), jnp.float32),
        pltpu.SemaphoreType.DMA,
        pltpu.SemaphoreType.DMA,
    ],
)
def _tscore(
    upx_h, upy_h, upz_h, uox_h, uoy_h, uoz_h, mp_h, mo_h, thp_h,
    score_h, hits_h, ll_h,
    upx, upy, upz, uox, uoy, uoz, mp, mo, thp,
    p_buf, c_buf, stage_p, stage_hh, stage_l, gbuf, outv, stage_sh, dsem, dsem2,
):
    ci = lax.axis_index("c")
    si = lax.axis_index("s")
    active = si < SEG_PER_CORE
    seg = ci * SEG_PER_CORE + si
    lane = lax.iota(jnp.int32, LANES)

    @pl.when(active)
    def _work():
        base = seg * ROW
        HALF = ROW // 2
        arrs = ((upx_h, upx), (upy_h, upy), (upz_h, upz), (uox_h, uox),
                (uoy_h, uoy), (uoz_h, uoz), (mp_h, mp), (mo_h, mo))
        sl0 = pl.ds(base, HALF)
        sl1 = pl.ds(base + HALF, HALF)
        cps0 = tuple(
            pltpu.async_copy(h.at[sl0], v.at[0], dsem)
            for h, v in arrs
        ) + (pltpu.async_copy(thp_h, thp, dsem),)
        cps1 = tuple(
            pltpu.async_copy(h.at[sl1], v.at[1], dsem2)
            for h, v in arrs
        )
        for cp in cps0:
            cp.wait()

        onehot = lane == seg
        th_all = _T_MIN * jnp.exp(thp[...] * _LOG_RANGE)
        one = jnp.float32(1.0)
        zero = jnp.float32(0.0)
        th = jnp.sum(jnp.where(onehot, th_all, zero))
        rinv = jnp.sum(jnp.where(onehot, one / th_all, zero))

        def chunk(h, i, accp, acch, accl, accn, acct):
            sl = pl.ds(i * LANES, LANES)
            slg = pl.ds(h * HALF + i * LANES, LANES)
            dux = upx[h, sl] - uox[h, sl]
            duy = upy[h, sl] - uoy[h, sl]
            duz = upz[h, sl] - uoz[h, sl]
            s2 = dux * dux + duy * duy + duz * duz
            close = s2 < th
            vv = s2 * rinv
            dm = mp[h, sl] - mo[h, sl]
            zz = dm * _INV_SIGMA
            pmag = _COEF * jnp.exp(np.float32(-0.5) * zz * zz)
            cf = jnp.where(close, one, zero)
            p = jnp.where(close, (one - vv) * pmag, zero)
            pos = p > zero
            selpos = jnp.where(close & pos, one, zero)
            lp = _logf(jnp.maximum(p, jnp.float32(1e-37)))
            tiny = jnp.where(close & pos & (p < jnp.float32(1e-26)), one, zero)
            p_buf[slg] = p
            c_buf[slg] = cf
            return (accp + p, acch + cf, accl + selpos * lp,
                    accn + selpos, acct + tiny)

        def mk_body(h):
            def body1(i, carry):
                a = chunk(h, 2 * i, *carry)
                return chunk(h, 2 * i + 1, *a)
            return body1

        zero16 = jnp.zeros((LANES,), jnp.float32)
        carry = lax.fori_loop(0, NITER // 4, mk_body(0), (zero16,) * 5)
        for cp in cps1:
            cp.wait()
        accp, acch, accl, accn, acct = lax.fori_loop(
            0, NITER // 4, mk_body(1), carry
        )
        ps = jnp.sum(accp)
        hs = jnp.sum(acch)
        npos = jnp.sum(accn)
        ntiny = jnp.sum(acct)
        den = jnp.maximum(ps, jnp.float32(1e-30))
        n0 = hs - npos
        logden = jnp.max(_logf(jnp.zeros((LANES,), jnp.float32) + den))
        ls_fast = jnp.sum(accl) - npos * logden + n0 * _LOG_1EM30

        def exact_ll(_):
            def body2(i, accl2):
                sl = pl.ds(i * LANES, LANES)
                t = jnp.maximum(p_buf[sl] / den, jnp.float32(1e-30))
                return accl2 + c_buf[sl] * _logf(t)

            return jnp.sum(lax.fori_loop(0, NITER, body2, zero16))

        ls = lax.cond(
            (ntiny > zero) | (den < jnp.float32(1e-6)),
            exact_ll,
            lambda _: ls_fast,
            zero,
        )

        stage_p[...] = jnp.where(onehot, ps, jnp.float32(0.0))
        stage_hh[...] = jnp.where(onehot, hs, jnp.float32(0.0))
        stage_l[...] = jnp.where(onehot, ls, jnp.float32(0.0))
        pltpu.sync_copy(stage_p, stage_sh.at[0, seg])
        pltpu.sync_copy(stage_hh, stage_sh.at[1, seg])
        pltpu.sync_copy(stage_l, stage_sh.at[2, seg])

    plsc.subcore_barrier()

    @pl.when(si == 0)
    def _gather():
        half = pl.ds(ci * SEG_PER_CORE, SEG_PER_CORE)
        for g, out_h in ((0, score_h), (1, hits_h), (2, ll_h)):
            pltpu.sync_copy(stage_sh.at[g, half], gbuf)
            acc = jnp.zeros((LANES,), jnp.float32)
            for i in range(SEG_PER_CORE):
                acc = acc + gbuf[i]
            outv[...] = acc
            pltpu.sync_copy(outv.at[half], out_h.at[half])


def kernel(u_pred, mag_pred, u_obs, mag_obs, thresh_s2_param):
    return _tscore(
        u_pred[:, 0], u_pred[:, 1], u_pred[:, 2],
        u_obs[:, 0], u_obs[:, 1], u_obs[:, 2],
        mag_pred, mag_obs, thresh_s2_param,
    )


# speedup vs baseline: 1.0640x; 1.0640x over previous
import functools
import math

import jax
import jax.numpy as jnp
import numpy as np
from jax import lax
from jax.experimental import pallas as pl
from jax.experimental.pallas import tpu as pltpu
from jax.experimental.pallas import tpu_sc as plsc

SPACE_DIMS = 3
N_SEG = 16
ROW = 2048
LANES = 16
NITER = ROW // LANES
NC = 2
NS = 16
SEG_PER_CORE = N_SEG // NC

def _deg2dist(deg):
    return 2.0 * np.sin(np.radians(np.asarray(deg, dtype=np.float64)) / 2.0)

_T_MIN = np.float32(_deg2dist(10.0 / 3600.0) ** 2)
_T_MAX = np.float32(_deg2dist(1.0) ** 2)
_LOG_RANGE = np.float32(np.log(np.float64(_T_MAX) / np.float64(_T_MIN)))
_SIGMA = np.float32(np.e)
_INV_SIGMA = np.float32(1.0) / _SIGMA
_COEF = np.float32(np.float32(1.0 / np.sqrt(2.0 * np.pi)) / _SIGMA)
_LN2 = np.float32(0.693147180559945309)
_LOG_1EM30 = np.float32(np.log(1e-30))


def _logf(x):
    ix = lax.bitcast_convert_type(x, jnp.int32)
    ix = ix + (0x3F800000 - 0x3F3504F3)
    k = lax.shift_right_arithmetic(ix, 23) - 127
    mx = (ix & 0x007FFFFF) + 0x3F3504F3
    m = lax.bitcast_convert_type(mx, jnp.float32)
    f = m - 1.0
    s = f / (2.0 + f)
    z = s * s
    w = z * z
    t1 = w * (np.float32(0.40000972152) + w * np.float32(0.24279078841))
    t2 = z * (np.float32(0.66666662693) + w * np.float32(0.28498786688))
    r = t2 + t1
    hfsq = np.float32(0.5) * f * f
    return f - (hfsq - s * (hfsq + r)) + k.astype(jnp.float32) * _LN2


@functools.partial(
    pl.kernel,
    out_type=(
        jax.ShapeDtypeStruct((N_SEG,), jnp.float32),
        jax.ShapeDtypeStruct((N_SEG,), jnp.float32),
        jax.ShapeDtypeStruct((N_SEG,), jnp.float32),
    ),
    mesh=plsc.VectorSubcoreMesh(
        core_axis_name="c", subcore_axis_name="s", num_cores=NC, num_subcores=NS
    ),
    compiler_params=pltpu.CompilerParams(needs_layout_passes=False),
    scratch_types=[
        pltpu.VMEM((ROW,), jnp.float32),
        pltpu.VMEM((ROW,), jnp.float32),
        pltpu.VMEM((ROW,), jnp.float32),
        pltpu.VMEM((ROW,), jnp.float32),
        pltpu.VMEM((ROW,), jnp.float32),
        pltpu.VMEM((ROW,), jnp.float32),
        pltpu.VMEM((ROW,), jnp.float32),
        pltpu.VMEM((ROW,), jnp.float32),
        pltpu.VMEM((LANES,), jnp.float32),
        pltpu.VMEM((ROW,), jnp.float32),
        pltpu.VMEM((ROW,), jnp.float32),
        pltpu.VMEM((LANES,), jnp.float32),
        pltpu.VMEM((LANES,), jnp.float32),
        pltpu.VMEM((LANES,), jnp.float32),
        pltpu.VMEM((SEG_PER_CORE, LANES), jnp.float32),
        pltpu.VMEM((LANES,), jnp.float32),
        pltpu.HBM((3, N_SEG, LANES), jnp.float32),
        pltpu.SemaphoreType.DMA,
    ],
)
def _tscore(
    upx_h, upy_h, upz_h, uox_h, uoy_h, uoz_h, mp_h, mo_h, thp_h,
    score_h, hits_h, ll_h,
    upx, upy, upz, uox, uoy, uoz, mp, mo, thp,
    p_buf, c_buf, stage_p, stage_hh, stage_l, gbuf, outv, stage_sh, dsem,
):
    ci = lax.axis_index("c")
    si = lax.axis_index("s")
    active = si < SEG_PER_CORE
    seg = ci * SEG_PER_CORE + si
    lane = lax.iota(jnp.int32, LANES)

    @pl.when(active)
    def _work():
        base = seg * ROW
        sl_h = pl.ds(base, ROW)
        cps = (
            pltpu.async_copy(upx_h.at[sl_h], upx, dsem),
            pltpu.async_copy(upy_h.at[sl_h], upy, dsem),
            pltpu.async_copy(upz_h.at[sl_h], upz, dsem),
            pltpu.async_copy(uox_h.at[sl_h], uox, dsem),
            pltpu.async_copy(uoy_h.at[sl_h], uoy, dsem),
            pltpu.async_copy(uoz_h.at[sl_h], uoz, dsem),
            pltpu.async_copy(mp_h.at[sl_h], mp, dsem),
            pltpu.async_copy(mo_h.at[sl_h], mo, dsem),
            pltpu.async_copy(thp_h, thp, dsem),
        )
        for cp in cps:
            cp.wait()

        onehot = lane == seg
        th_all = _T_MIN * jnp.exp(thp[...] * _LOG_RANGE)
        one = jnp.float32(1.0)
        zero = jnp.float32(0.0)
        th = jnp.sum(jnp.where(onehot, th_all, zero))
        rinv = jnp.sum(jnp.where(onehot, one / th_all, zero))

        def chunk(i, accp, acch, accl, accn, acct):
            sl = pl.ds(i * LANES, LANES)
            dux = upx[sl] - uox[sl]
            duy = upy[sl] - uoy[sl]
            duz = upz[sl] - uoz[sl]
            s2 = dux * dux + duy * duy + duz * duz
            close = s2 < th
            vv = s2 * rinv
            dm = mp[sl] - mo[sl]
            zz = dm * _INV_SIGMA
            pmag = _COEF * jnp.exp(np.float32(-0.5) * zz * zz)
            cf = jnp.where(close, one, zero)
            p = jnp.where(close, (one - vv) * pmag, zero)
            pos = p > zero
            selpos = jnp.where(close & pos, one, zero)
            lp = _logf(jnp.maximum(p, jnp.float32(1e-37)))
            tiny = jnp.where(close & pos & (p < jnp.float32(1e-26)), one, zero)
            p_buf[sl] = p
            c_buf[sl] = cf
            return (accp + p, acch + cf, accl + selpos * lp,
                    accn + selpos, acct + tiny)

        def body1(i, carry):
            a = chunk(2 * i, *carry)
            return chunk(2 * i + 1, *a)

        zero16 = jnp.zeros((LANES,), jnp.float32)
        accp, acch, accl, accn, acct = lax.fori_loop(
            0, NITER // 2, body1, (zero16,) * 5
        )
        ps = jnp.sum(accp)
        hs = jnp.sum(acch)
        npos = jnp.sum(accn)
        ntiny = jnp.sum(acct)
        den = jnp.maximum(ps, jnp.float32(1e-30))
        n0 = hs - npos
        logden = jnp.max(_logf(jnp.zeros((LANES,), jnp.float32) + den))
        ls_fast = jnp.sum(accl) - npos * logden + n0 * _LOG_1EM30

        def exact_ll(_):
            def body2(i, accl2):
                sl = pl.ds(i * LANES, LANES)
                t = jnp.maximum(p_buf[sl] / den, jnp.float32(1e-30))
                return accl2 + c_buf[sl] * _logf(t)

            return jnp.sum(lax.fori_loop(0, NITER, body2, zero16))

        ls = lax.cond(
            (ntiny > zero) | (den < jnp.float32(1e-6)),
            exact_ll,
            lambda _: ls_fast,
            zero,
        )

        stage_p[...] = jnp.where(onehot, ps, jnp.float32(0.0))
        stage_hh[...] = jnp.where(onehot, hs, jnp.float32(0.0))
        stage_l[...] = jnp.where(onehot, ls, jnp.float32(0.0))
        pltpu.sync_copy(stage_p, stage_sh.at[0, seg])
        pltpu.sync_copy(stage_hh, stage_sh.at[1, seg])
        pltpu.sync_copy(stage_l, stage_sh.at[2, seg])

    plsc.subcore_barrier()

    @pl.when(si == 0)
    def _gather():
        half = pl.ds(ci * SEG_PER_CORE, SEG_PER_CORE)
        for g, out_h in ((0, score_h), (1, hits_h), (2, ll_h)):
            pltpu.sync_copy(stage_sh.at[g, half], gbuf)
            acc = jnp.zeros((LANES,), jnp.float32)
            for i in range(SEG_PER_CORE):
                acc = acc + gbuf[i]
            outv[...] = acc
            pltpu.sync_copy(outv.at[half], out_h.at[half])


def kernel(u_pred, mag_pred, u_obs, mag_obs, thresh_s2_param):
    return _tscore(
        u_pred[:, 0], u_pred[:, 1], u_pred[:, 2],
        u_obs[:, 0], u_obs[:, 1], u_obs[:, 2],
        mag_pred, mag_obs, thresh_s2_param,
    )


# speedup vs baseline: 1.0999x; 1.0337x over previous
import functools
import math

import jax
import jax.numpy as jnp
import numpy as np
from jax import lax
from jax.experimental import pallas as pl
from jax.experimental.pallas import tpu as pltpu
from jax.experimental.pallas import tpu_sc as plsc

SPACE_DIMS = 3
N_SEG = 16
ROW = 2048
LANES = 16
NITER = ROW // LANES
NC = 2
NS = 16
SEG_PER_CORE = N_SEG // NC

def _deg2dist(deg):
    return 2.0 * np.sin(np.radians(np.asarray(deg, dtype=np.float64)) / 2.0)

_T_MIN = np.float32(_deg2dist(10.0 / 3600.0) ** 2)
_T_MAX = np.float32(_deg2dist(1.0) ** 2)
_LOG_RANGE = np.float32(np.log(np.float64(_T_MAX) / np.float64(_T_MIN)))
_SIGMA = np.float32(np.e)
_INV_SIGMA = np.float32(1.0) / _SIGMA
_COEF = np.float32(np.float32(1.0 / np.sqrt(2.0 * np.pi)) / _SIGMA)
_LN2 = np.float32(0.693147180559945309)
_LOG_1EM30 = np.float32(np.log(1e-30))


def _logf(x):
    ix = lax.bitcast_convert_type(x, jnp.int32)
    ix = ix + (0x3F800000 - 0x3F3504F3)
    k = lax.shift_right_arithmetic(ix, 23) - 127
    mx = (ix & 0x007FFFFF) + 0x3F3504F3
    m = lax.bitcast_convert_type(mx, jnp.float32)
    f = m - 1.0
    s = f / (2.0 + f)
    z = s * s
    w = z * z
    t1 = w * (np.float32(0.40000972152) + w * np.float32(0.24279078841))
    t2 = z * (np.float32(0.66666662693) + w * np.float32(0.28498786688))
    r = t2 + t1
    hfsq = np.float32(0.5) * f * f
    return f - (hfsq - s * (hfsq + r)) + k.astype(jnp.float32) * _LN2


@functools.partial(
    pl.kernel,
    out_type=(
        jax.ShapeDtypeStruct((N_SEG,), jnp.float32),
        jax.ShapeDtypeStruct((N_SEG,), jnp.float32),
        jax.ShapeDtypeStruct((N_SEG,), jnp.float32),
    ),
    mesh=plsc.VectorSubcoreMesh(
        core_axis_name="c", subcore_axis_name="s", num_cores=NC, num_subcores=NS
    ),
    compiler_params=pltpu.CompilerParams(needs_layout_passes=False),
    scratch_types=[
        pltpu.VMEM((ROW,), jnp.float32),
        pltpu.VMEM((ROW,), jnp.float32),
        pltpu.VMEM((ROW,), jnp.float32),
        pltpu.VMEM((ROW,), jnp.float32),
        pltpu.VMEM((ROW,), jnp.float32),
        pltpu.VMEM((ROW,), jnp.float32),
        pltpu.VMEM((ROW,), jnp.float32),
        pltpu.VMEM((ROW,), jnp.float32),
        pltpu.VMEM((LANES,), jnp.float32),
        pltpu.VMEM((LANES,), jnp.float32),
        pltpu.VMEM((LANES,), jnp.float32),
        pltpu.VMEM((LANES,), jnp.float32),
        pltpu.VMEM((3, SEG_PER_CORE, LANES), jnp.float32),
        pltpu.VMEM((LANES,), jnp.float32),
        pltpu.HBM((3, N_SEG, LANES), jnp.float32),
        pltpu.SemaphoreType.DMA,
    ],
)
def _tscore(
    upx_h, upy_h, upz_h, uox_h, uoy_h, uoz_h, mp_h, mo_h, thp_h,
    score_h, hits_h, ll_h,
    upx, upy, upz, uox, uoy, uoz, mp, mo, thp,
    stage_p, stage_hh, stage_l, gbuf, outv, stage_sh, dsem,
):
    ci = lax.axis_index("c")
    si = lax.axis_index("s")
    active = si < SEG_PER_CORE
    seg = ci * SEG_PER_CORE + si
    lane = lax.iota(jnp.int32, LANES)

    @pl.when(active)
    def _work():
        base = seg * ROW
        sl_h = pl.ds(base, ROW)
        cps = (
            pltpu.async_copy(upx_h.at[sl_h], upx, dsem),
            pltpu.async_copy(upy_h.at[sl_h], upy, dsem),
            pltpu.async_copy(upz_h.at[sl_h], upz, dsem),
            pltpu.async_copy(uox_h.at[sl_h], uox, dsem),
            pltpu.async_copy(uoy_h.at[sl_h], uoy, dsem),
            pltpu.async_copy(uoz_h.at[sl_h], uoz, dsem),
            pltpu.async_copy(mp_h.at[sl_h], mp, dsem),
            pltpu.async_copy(mo_h.at[sl_h], mo, dsem),
            pltpu.async_copy(thp_h, thp, dsem),
        )
        for cp in cps:
            cp.wait()

        onehot = lane == seg
        th_all = _T_MIN * jnp.exp(thp[...] * _LOG_RANGE)
        one = jnp.float32(1.0)
        zero = jnp.float32(0.0)
        th = jnp.sum(jnp.where(onehot, th_all, zero))
        rinv = jnp.sum(jnp.where(onehot, one / th_all, zero))

        def chunk(i, accp, acch, accl, accn, acct):
            sl = pl.ds(i * LANES, LANES)
            dux = upx[sl] - uox[sl]
            duy = upy[sl] - uoy[sl]
            duz = upz[sl] - uoz[sl]
            s2 = dux * dux + duy * duy + duz * duz
            close = s2 < th
            vv = s2 * rinv
            dm = mp[sl] - mo[sl]
            zz = dm * _INV_SIGMA
            pmag = _COEF * jnp.exp(np.float32(-0.5) * zz * zz)
            cf = jnp.where(close, one, zero)
            p = jnp.where(close, (one - vv) * pmag, zero)
            pos = p > zero
            selpos = jnp.where(close & pos, one, zero)
            lp = _logf(jnp.maximum(p, jnp.float32(1e-37)))
            tiny = jnp.where(close & pos & (p < jnp.float32(1e-26)), one, zero)
            return (accp + p, acch + cf, accl + selpos * lp,
                    accn + selpos, acct + tiny)

        def body1(i, carry):
            a = chunk(4 * i, *carry)
            a = chunk(4 * i + 1, *a)
            a = chunk(4 * i + 2, *a)
            return chunk(4 * i + 3, *a)

        zero16 = jnp.zeros((LANES,), jnp.float32)
        accp, acch, accl, accn, acct = lax.fori_loop(
            0, NITER // 4, body1, (zero16,) * 5
        )
        ps = jnp.sum(accp)
        hs = jnp.sum(acch)
        npos = jnp.sum(accn)
        ntiny = jnp.sum(acct)
        den = jnp.maximum(ps, jnp.float32(1e-30))
        n0 = hs - npos
        logden = jnp.max(_logf(jnp.zeros((LANES,), jnp.float32) + den))
        ls_fast = jnp.sum(accl) - npos * logden + n0 * _LOG_1EM30

        def exact_ll(_):
            def body2(i, accl2):
                sl = pl.ds(i * LANES, LANES)
                dux = upx[sl] - uox[sl]
                duy = upy[sl] - uoy[sl]
                duz = upz[sl] - uoz[sl]
                s2 = dux * dux + duy * duy + duz * duz
                close = s2 < th
                dm = mp[sl] - mo[sl]
                zz = dm * _INV_SIGMA
                pmag = _COEF * jnp.exp(np.float32(-0.5) * zz * zz)
                p = jnp.where(close, (one - s2 * rinv) * pmag, zero)
                cf = jnp.where(close, one, zero)
                t = jnp.maximum(p / den, jnp.float32(1e-30))
                return accl2 + cf * _logf(t)

            return jnp.sum(lax.fori_loop(0, NITER, body2, zero16))

        ls = lax.cond(
            (ntiny > zero) | (den < jnp.float32(1e-6)),
            exact_ll,
            lambda _: ls_fast,
            zero,
        )

        stage_p[...] = jnp.where(onehot, ps, jnp.float32(0.0))
        stage_hh[...] = jnp.where(onehot, hs, jnp.float32(0.0))
        stage_l[...] = jnp.where(onehot, ls, jnp.float32(0.0))
        pltpu.sync_copy(stage_p, stage_sh.at[0, seg])
        pltpu.sync_copy(stage_hh, stage_sh.at[1, seg])
        pltpu.sync_copy(stage_l, stage_sh.at[2, seg])

    plsc.subcore_barrier()

    @pl.when(si == 0)
    def _gather():
        half = pl.ds(ci * SEG_PER_CORE, SEG_PER_CORE)
        cpg = tuple(
            pltpu.async_copy(stage_sh.at[g, half], gbuf.at[g], dsem)
            for g in range(3)
        )
        for cp in cpg:
            cp.wait()
        for g, out_h in ((0, score_h), (1, hits_h), (2, ll_h)):
            acc = jnp.zeros((LANES,), jnp.float32)
            for i in range(SEG_PER_CORE):
                acc = acc + gbuf[g, i]
            outv[...] = acc
            pltpu.sync_copy(outv.at[half], out_h.at[half])


def kernel(u_pred, mag_pred, u_obs, mag_obs, thresh_s2_param):
    return _tscore(
        u_pred[:, 0], u_pred[:, 1], u_pred[:, 2],
        u_obs[:, 0], u_obs[:, 1], u_obs[:, 2],
        mag_pred, mag_obs, thresh_s2_param,
    )
